# baseline (speedup 1.0000x reference)
import functools
import jax
import jax.numpy as jnp
from jax import lax
from jax.experimental import pallas as pl
from jax.experimental.pallas import tpu as pltpu
from jax.experimental.pallas import tpu_sc as plsc

_N = 10000
_H = 128
_E = 320000
_R = 4
_NC = 2
_NS = 16
_NW = _NC * _NS
_EPW = _E // _NW
_C = 80
_EPWP = 10080
_NCHP = _EPWP // _C
_AGGR = 10016
_ZR = 624
_ZC = 80
_DSHIFT = 14


def _sc_agg_body(ycat, pk, zin, out,
                 pk_v, gi0, gi1, db0, db1, rb0, rb1, agg,
                 semE, sem0, sem1):
    cid = lax.axis_index("c")
    sid = lax.axis_index("s")
    wid = sid * _NC + cid
    ebase = wid * _EPWP

    stg = pltpu.async_copy(pk.at[pl.ds(ebase, _EPWP)], pk_v, semE)

    pltpu.sync_copy(zin, rb0)
    zbase = sid * _ZR
    for t in range(7):
        pltpu.sync_copy(rb0.at[pl.ds(0, _ZC)], agg.at[pl.ds(zbase + t * _ZC, _ZC)])
    pltpu.sync_copy(rb0.at[pl.ds(0, 64)], agg.at[pl.ds(zbase + 560, 64)])

    @pl.when(sid == _NS - 1)
    def _():
        pltpu.sync_copy(rb0.at[pl.ds(0, 32)], agg.at[pl.ds(_NS * _ZR, 32)])

    plsc.subcore_barrier()
    stg.wait()

    def _prep(j, gi, db):
        base = j * _C
        for k in range(_C // 16):
            s = pl.ds(base + k * 16, 16)
            w = pk_v[s]
            gi[pl.ds(k * 16, 16)] = lax.shift_right_logical(w, _DSHIFT)
            db[pl.ds(k * 16, 16)] = lax.bitwise_and(w, (1 << _DSHIFT) - 1)

    def _fire(gi, rb, sem):
        pltpu.async_copy(ycat.at[gi], rb, sem)

    def _wait(gi, rb, sem):
        pltpu.make_async_copy(ycat.at[gi], rb, sem).wait()

    _prep(0, gi0, db0)
    _fire(gi0, rb0, sem0)

    def _pair(jj, carry):
        j = jj * 2
        _prep(j + 1, gi1, db1)
        _fire(gi1, rb1, sem1)
        _wait(gi0, rb0, sem0)
        pltpu.sync_copy(rb0, agg.at[db0], add=True)

        @pl.when(j + 2 < _NCHP)
        def _():
            _prep(j + 2, gi0, db0)
            _fire(gi0, rb0, sem0)

        _wait(gi1, rb1, sem1)
        pltpu.sync_copy(rb1, agg.at[db1], add=True)
        return carry

    lax.fori_loop(0, _NCHP // 2, _pair, 0)

    plsc.subcore_barrier()

    RB = (rb0, rb1)
    SD = (sem0, sem1)
    descs = []
    for t in range(8):
        b = t % 2
        rows = _ZC if t < 7 else 64
        if t >= 2:
            descs[t - 2].wait()
        pltpu.sync_copy(agg.at[pl.ds(zbase + t * _ZC, rows)],
                        RB[b].at[pl.ds(0, rows)])
        descs.append(pltpu.async_copy(
            RB[b].at[pl.ds(0, rows)],
            out.at[cid, pl.ds(zbase + t * _ZC, rows)], SD[b]))
    descs[6].wait()
    descs[7].wait()

    @pl.when(sid == _NS - 1)
    def _():
        pltpu.sync_copy(agg.at[pl.ds(_NS * _ZR, 16)], rb0.at[pl.ds(0, 16)])
        pltpu.sync_copy(rb0.at[pl.ds(0, 16)], out.at[cid, pl.ds(_NS * _ZR, 16)])


_sc_agg = pl.kernel(
    _sc_agg_body,
    out_type=jax.ShapeDtypeStruct((_NC, _N, _H), jnp.float32),
    mesh=plsc.VectorSubcoreMesh(
        core_axis_name="c", subcore_axis_name="s",
        num_cores=_NC, num_subcores=_NS),
    scratch_types=[
        pltpu.VMEM((_EPWP,), jnp.int32),
        pltpu.VMEM((_C,), jnp.int32),
        pltpu.VMEM((_C,), jnp.int32),
        pltpu.VMEM((_C,), jnp.int32),
        pltpu.VMEM((_C,), jnp.int32),
        pltpu.VMEM((_C, _H), jnp.float32),
        pltpu.VMEM((_C, _H), jnp.float32),
        pltpu.VMEM_SHARED((_AGGR, _H), jnp.float32),
        pltpu.SemaphoreType.DMA,
        pltpu.SemaphoreType.DMA,
        pltpu.SemaphoreType.DMA,
    ],
)


def _leaky(x):
    return jnp.where(x > 0, x, 0.3 * x)


def _mlp2(x, W1, b1, W2, b2, bm):
    M, K = x.shape
    N1 = W1.shape[-1]
    N2 = W2.shape[-1]

    def body(xr, w1r, b1r, w2r, b2r, orf):
        h = jnp.dot(xr[...], w1r[...], preferred_element_type=jnp.float32)
        h = _leaky(h + b1r[...])
        orf[...] = jnp.dot(h, w2r[...], preferred_element_type=jnp.float32) + b2r[...]

    return pl.pallas_call(
        body,
        grid=(M // bm,),
        in_specs=[
            pl.BlockSpec((bm, K), lambda i: (i, 0)),
            pl.BlockSpec((K, N1), lambda i: (0, 0)),
            pl.BlockSpec((1, N1), lambda i: (0, 0)),
            pl.BlockSpec((N1, N2), lambda i: (0, 0)),
            pl.BlockSpec((1, N2), lambda i: (0, 0)),
        ],
        out_specs=pl.BlockSpec((bm, N2), lambda i: (i, 0)),
        out_shape=jax.ShapeDtypeStruct((M, N2), jnp.float32),
    )(x, W1, b1.reshape(1, N1), W2, b2.reshape(1, N2))


def _lin(x, W, bm):
    M, K = x.shape
    N = W.shape[-1]

    def body(xr, wr, orf):
        orf[...] = jnp.dot(xr[...], wr[...], preferred_element_type=jnp.float32)

    return pl.pallas_call(
        body,
        grid=(M // bm,),
        in_specs=[
            pl.BlockSpec((bm, K), lambda i: (i, 0)),
            pl.BlockSpec((K, N), lambda i: (0, 0)),
        ],
        out_specs=pl.BlockSpec((bm, N), lambda i: (i, 0)),
        out_shape=jax.ShapeDtypeStruct((M, N), jnp.float32),
    )(x, W)


def _edge_prep(src2d, et2d, dst2d):
    rows, cols = src2d.shape

    def body(sr, er, dr, orf):
        gi = sr[...] * _R + er[...]
        orf[...] = jnp.bitwise_or(jnp.left_shift(gi, _DSHIFT), dr[...])

    return pl.pallas_call(
        body,
        grid=(1,),
        in_specs=[
            pl.BlockSpec((rows, cols), lambda i: (0, 0)),
            pl.BlockSpec((rows, cols), lambda i: (0, 0)),
            pl.BlockSpec((rows, cols), lambda i: (0, 0)),
        ],
        out_specs=pl.BlockSpec((rows, cols), lambda i: (0, 0)),
        out_shape=jax.ShapeDtypeStruct((rows, cols), jnp.int32),
    )(src2d, et2d, dst2d)


def _linb(x, W, b, bm):
    M, K = x.shape
    N = W.shape[-1]

    def body(xr, wr, br, orf):
        orf[...] = jnp.dot(xr[...], wr[...], preferred_element_type=jnp.float32) + br[...]

    return pl.pallas_call(
        body,
        grid=(M // bm,),
        in_specs=[
            pl.BlockSpec((bm, K), lambda i: (i, 0)),
            pl.BlockSpec((K, N), lambda i: (0, 0)),
            pl.BlockSpec((1, N), lambda i: (0, 0)),
        ],
        out_specs=pl.BlockSpec((bm, N), lambda i: (i, 0)),
        out_shape=jax.ShapeDtypeStruct((M, N), jnp.float32),
    )(x, W, b.reshape(1, N))


def _combine_y(a0, a1, sl, wcat, bm):
    M, K = sl.shape
    NY = wcat.shape[-1]

    def body(a0r, a1r, slr, wr, f_ref, y_ref):
        f = a0r[...] + a1r[...] + slr[...]
        f_ref[...] = f
        y_ref[...] = jnp.dot(f, wr[...], preferred_element_type=jnp.float32)

    return pl.pallas_call(
        body,
        grid=(M // bm,),
        in_specs=[
            pl.BlockSpec((bm, K), lambda i: (i, 0)),
            pl.BlockSpec((bm, K), lambda i: (i, 0)),
            pl.BlockSpec((bm, K), lambda i: (i, 0)),
            pl.BlockSpec((K, NY), lambda i: (0, 0)),
        ],
        out_specs=[
            pl.BlockSpec((bm, K), lambda i: (i, 0)),
            pl.BlockSpec((bm, NY), lambda i: (i, 0)),
        ],
        out_shape=[
            jax.ShapeDtypeStruct((M, K), jnp.float32),
            jax.ShapeDtypeStruct((M, NY), jnp.float32),
        ],
    )(a0, a1, sl, wcat)


def _decode(a0, a1, sl, dW1, db1, dW2, db2, ceW1, ceb1, ceW2, ceb2,
            cdW1, cdb1, cdW2, cdb2, bm):
    half = a0.shape[0] // 2
    ng = half // bm
    nclass = dW2.shape[-1]

    def body(a0a, a1a, sla, a0b, a1b, slb, dw1, db1r, dw2, db2r,
             cw1, cb1, cw2, cb2, kw1, kb1, kw2, kb2, res_ref, loss_ref):
        def dec(x):
            h = _leaky(jnp.dot(x, dw1[...], preferred_element_type=jnp.float32) + db1r[...])
            return jnp.dot(h, dw2[...], preferred_element_type=jnp.float32) + db2r[...]

        def constr(f):
            h = _leaky(jnp.dot(f, cw1[...], preferred_element_type=jnp.float32) + cb1[...])
            c = jnp.dot(h, cw2[...], preferred_element_type=jnp.float32) + cb2[...]
            h2 = _leaky(jnp.dot(c, kw1[...], preferred_element_type=jnp.float32) + kb1[...])
            return jnp.dot(h2, kw2[...], preferred_element_type=jnp.float32) + kb2[...]

        fa = dec(a0a[...] + a1a[...] + sla[...])
        fb = dec(a0b[...] + a1b[...] + slb[...])
        ca = constr(fa)
        cb = constr(fb)
        res_ref[...] = 0.5 * (ca + cb)
        part = jnp.sum((ca - fa) ** 2) + jnp.sum((cb - fb) ** 2)
        i = pl.program_id(0)

        @pl.when(i == 0)
        def _():
            loss_ref[...] = jnp.zeros((1, 1), jnp.float32)

        loss_ref[...] += part.reshape(1, 1)

        @pl.when(i == ng - 1)
        def _():
            loss_ref[...] = loss_ref[...] / (2.0 * half * nclass)

    def full(shape):
        return [pl.BlockSpec(shape, lambda i: (0, 0))]

    res, loss = pl.pallas_call(
        body,
        grid=(ng,),
        in_specs=(
            [pl.BlockSpec((bm, a0.shape[1]), lambda i: (i, 0)),
             pl.BlockSpec((bm, a0.shape[1]), lambda i: (i, 0)),
             pl.BlockSpec((bm, a0.shape[1]), lambda i: (i, 0)),
             pl.BlockSpec((bm, a0.shape[1]), lambda i: (i + ng, 0)),
             pl.BlockSpec((bm, a0.shape[1]), lambda i: (i + ng, 0)),
             pl.BlockSpec((bm, a0.shape[1]), lambda i: (i + ng, 0))]
            + full(dW1.shape) + full((1, dW1.shape[1]))
            + full(dW2.shape) + full((1, dW2.shape[1]))
            + full(ceW1.shape) + full((1, ceW1.shape[1]))
            + full(ceW2.shape) + full((1, ceW2.shape[1]))
            + full(cdW1.shape) + full((1, cdW1.shape[1]))
            + full(cdW2.shape) + full((1, cdW2.shape[1]))
        ),
        out_specs=[
            pl.BlockSpec((bm, nclass), lambda i: (i, 0)),
            pl.BlockSpec((1, 1), lambda i: (0, 0)),
        ],
        out_shape=[
            jax.ShapeDtypeStruct((half, nclass), jnp.float32),
            jax.ShapeDtypeStruct((1, 1), jnp.float32),
        ],
    )(a0, a1, sl, a0, a1, sl, dW1, db1.reshape(1, -1), dW2, db2.reshape(1, -1),
      ceW1, ceb1.reshape(1, -1), ceW2, ceb2.reshape(1, -1),
      cdW1, cdb1.reshape(1, -1), cdW2, cdb2.reshape(1, -1))
    return res, loss[0, 0]


def kernel(feat0, feat1, edge_index, etypes,
           e0_W1, e0_b1, e0_W2, e0_b2,
           e1_W1, e1_b1, e1_W2, e1_b2,
           rel_W, self_W, rel_b,
           dec_W1, dec_b1, dec_W2, dec_b2,
           ce_W1, ce_b1, ce_W2, ce_b2,
           cd_W1, cd_b1, cd_W2, cd_b2):
    h0 = _mlp2(feat0, e0_W1, e0_b1, e0_W2, e0_b2, bm=1000)
    h1 = _mlp2(feat1, e1_W1, e1_b1, e1_W2, e1_b2, bm=1000)
    feat = jnp.concatenate([h0, h1], axis=0)

    src = edge_index[0].astype(jnp.int32)
    dst = edge_index[1].astype(jnp.int32)
    et = etypes.astype(jnp.int32)
    pk = _edge_prep(src.reshape(_E // _H, _H), et.reshape(_E // _H, _H),
                    dst.reshape(_E // _H, _H))
    pad = _EPWP - _EPW
    pk = jnp.pad(pk.reshape(_NW, _EPW), ((0, 0), (0, pad)),
                 constant_values=_N).reshape(-1)
    zin = jnp.zeros((_C, _H), jnp.float32)

    wcat0 = jnp.transpose(rel_W[0], (1, 0, 2)).reshape(_H, _R * _H)
    wcat1 = jnp.transpose(rel_W[1], (1, 0, 2)).reshape(_H, _R * _H)

    y0 = _lin(feat, wcat0, bm=2000).reshape(_N * _R, _H)
    agg_0 = _sc_agg(y0, pk, zin)
    sl0 = _linb(feat, self_W[0], rel_b[0], bm=2000)

    feat1, y1 = _combine_y(agg_0[0], agg_0[1], sl0, wcat1, bm=2000)
    agg_1 = _sc_agg(y1.reshape(_N * _R, _H), pk, zin)
    sl1 = _linb(feat1, self_W[1], rel_b[1], bm=2000)

    res, loss = _decode(agg_1[0], agg_1[1], sl1,
                        dec_W1, dec_b1, dec_W2, dec_b2,
                        ce_W1, ce_b1, ce_W2, ce_b2,
                        cd_W1, cd_b1, cd_W2, cd_b2, bm=1000)
    return res, loss


# speedup vs baseline: 1.0005x; 1.0005x over previous
import functools
import jax
import jax.numpy as jnp
from jax import lax
from jax.experimental import pallas as pl
from jax.experimental.pallas import tpu as pltpu
from jax.experimental.pallas import tpu_sc as plsc

_N = 10000
_H = 128
_E = 320000
_R = 4
_NC = 2
_NS = 16
_NW = _NC * _NS
_EPW = _E // _NW
_C = 80
_EPWP = 10080
_NCHP = _EPWP // _C
_AGGR = 10016
_ZR = 624
_ZC = 80
_DSHIFT = 14


def _sc_agg_body(ycat, pk, zin, out,
                 pk_v, gi0, gi1, db0, db1, rb0, rb1, agg,
                 semE, sem0, sem1):
    cid = lax.axis_index("c")
    sid = lax.axis_index("s")
    wid = sid * _NC + cid
    ebase = wid * _EPWP

    stg = pltpu.async_copy(pk.at[pl.ds(ebase, _EPWP)], pk_v, semE)

    pltpu.sync_copy(zin, rb0)
    zbase = sid * _ZR
    for t in range(7):
        pltpu.sync_copy(rb0.at[pl.ds(0, _ZC)], agg.at[pl.ds(zbase + t * _ZC, _ZC)])
    pltpu.sync_copy(rb0.at[pl.ds(0, 64)], agg.at[pl.ds(zbase + 560, 64)])

    @pl.when(sid == _NS - 1)
    def _():
        pltpu.sync_copy(rb0.at[pl.ds(0, 32)], agg.at[pl.ds(_NS * _ZR, 32)])

    plsc.subcore_barrier()
    stg.wait()

    def _prep(j, gi, db):
        base = j * _C
        for k in range(_C // 16):
            s = pl.ds(base + k * 16, 16)
            w = pk_v[s]
            gi[pl.ds(k * 16, 16)] = lax.shift_right_logical(w, _DSHIFT)
            db[pl.ds(k * 16, 16)] = lax.bitwise_and(w, (1 << _DSHIFT) - 1)

    def _fire(gi, rb, sem):
        pltpu.async_copy(ycat.at[gi], rb, sem)

    def _wait(gi, rb, sem):
        pltpu.make_async_copy(ycat.at[gi], rb, sem).wait()

    _prep(0, gi0, db0)
    _fire(gi0, rb0, sem0)

    def _pair(jj, carry):
        j = jj * 2
        _prep(j + 1, gi1, db1)
        _fire(gi1, rb1, sem1)
        _wait(gi0, rb0, sem0)
        pltpu.sync_copy(rb0, agg.at[db0], add=True)

        @pl.when(j + 2 < _NCHP)
        def _():
            _prep(j + 2, gi0, db0)
            _fire(gi0, rb0, sem0)

        _wait(gi1, rb1, sem1)
        pltpu.sync_copy(rb1, agg.at[db1], add=True)
        return carry

    lax.fori_loop(0, _NCHP // 2, _pair, 0)

    plsc.subcore_barrier()

    RB = (rb0, rb1)
    SD = (sem0, sem1)
    descs = []
    for t in range(8):
        b = t % 2
        rows = _ZC if t < 7 else 64
        if t >= 2:
            descs[t - 2].wait()
        pltpu.sync_copy(agg.at[pl.ds(zbase + t * _ZC, rows)],
                        RB[b].at[pl.ds(0, rows)])
        descs.append(pltpu.async_copy(
            RB[b].at[pl.ds(0, rows)],
            out.at[cid, pl.ds(zbase + t * _ZC, rows)], SD[b]))
    descs[6].wait()
    descs[7].wait()

    @pl.when(sid == _NS - 1)
    def _():
        pltpu.sync_copy(agg.at[pl.ds(_NS * _ZR, 16)], rb0.at[pl.ds(0, 16)])
        pltpu.sync_copy(rb0.at[pl.ds(0, 16)], out.at[cid, pl.ds(_NS * _ZR, 16)])


_sc_agg = pl.kernel(
    _sc_agg_body,
    out_type=jax.ShapeDtypeStruct((_NC, _N, _H), jnp.float32),
    mesh=plsc.VectorSubcoreMesh(
        core_axis_name="c", subcore_axis_name="s",
        num_cores=_NC, num_subcores=_NS),
    scratch_types=[
        pltpu.VMEM((_EPWP,), jnp.int32),
        pltpu.VMEM((_C,), jnp.int32),
        pltpu.VMEM((_C,), jnp.int32),
        pltpu.VMEM((_C,), jnp.int32),
        pltpu.VMEM((_C,), jnp.int32),
        pltpu.VMEM((_C, _H), jnp.float32),
        pltpu.VMEM((_C, _H), jnp.float32),
        pltpu.VMEM_SHARED((_AGGR, _H), jnp.float32),
        pltpu.SemaphoreType.DMA,
        pltpu.SemaphoreType.DMA,
        pltpu.SemaphoreType.DMA,
    ],
)


def _leaky(x):
    return jnp.where(x > 0, x, 0.3 * x)


def _mlp2(x, W1, b1, W2, b2, bm):
    M, K = x.shape
    N1 = W1.shape[-1]
    N2 = W2.shape[-1]

    def body(xr, w1r, b1r, w2r, b2r, orf):
        h = jnp.dot(xr[...], w1r[...], preferred_element_type=jnp.float32)
        h = _leaky(h + b1r[...])
        orf[...] = jnp.dot(h, w2r[...], preferred_element_type=jnp.float32) + b2r[...]

    return pl.pallas_call(
        body,
        grid=(M // bm,),
        in_specs=[
            pl.BlockSpec((bm, K), lambda i: (i, 0)),
            pl.BlockSpec((K, N1), lambda i: (0, 0)),
            pl.BlockSpec((1, N1), lambda i: (0, 0)),
            pl.BlockSpec((N1, N2), lambda i: (0, 0)),
            pl.BlockSpec((1, N2), lambda i: (0, 0)),
        ],
        out_specs=pl.BlockSpec((bm, N2), lambda i: (i, 0)),
        out_shape=jax.ShapeDtypeStruct((M, N2), jnp.float32),
    )(x, W1, b1.reshape(1, N1), W2, b2.reshape(1, N2))


def _lin(x, W, bm):
    M, K = x.shape
    N = W.shape[-1]

    def body(xr, wr, orf):
        orf[...] = jnp.dot(xr[...], wr[...], preferred_element_type=jnp.float32)

    return pl.pallas_call(
        body,
        grid=(M // bm,),
        in_specs=[
            pl.BlockSpec((bm, K), lambda i: (i, 0)),
            pl.BlockSpec((K, N), lambda i: (0, 0)),
        ],
        out_specs=pl.BlockSpec((bm, N), lambda i: (i, 0)),
        out_shape=jax.ShapeDtypeStruct((M, N), jnp.float32),
    )(x, W)


def _edge_prep(src2d, et2d, dst2d):
    rows, cols = src2d.shape

    def body(sr, er, dr, orf):
        gi = sr[...] * _R + er[...]
        orf[...] = jnp.bitwise_or(jnp.left_shift(gi, _DSHIFT), dr[...])

    return pl.pallas_call(
        body,
        grid=(1,),
        in_specs=[
            pl.BlockSpec((rows, cols), lambda i: (0, 0)),
            pl.BlockSpec((rows, cols), lambda i: (0, 0)),
            pl.BlockSpec((rows, cols), lambda i: (0, 0)),
        ],
        out_specs=pl.BlockSpec((rows, cols), lambda i: (0, 0)),
        out_shape=jax.ShapeDtypeStruct((rows, cols), jnp.int32),
    )(src2d, et2d, dst2d)


def _linb(x, W, b, bm):
    M, K = x.shape
    N = W.shape[-1]

    def body(xr, wr, br, orf):
        orf[...] = jnp.dot(xr[...], wr[...], preferred_element_type=jnp.float32) + br[...]

    return pl.pallas_call(
        body,
        grid=(M // bm,),
        in_specs=[
            pl.BlockSpec((bm, K), lambda i: (i, 0)),
            pl.BlockSpec((K, N), lambda i: (0, 0)),
            pl.BlockSpec((1, N), lambda i: (0, 0)),
        ],
        out_specs=pl.BlockSpec((bm, N), lambda i: (i, 0)),
        out_shape=jax.ShapeDtypeStruct((M, N), jnp.float32),
    )(x, W, b.reshape(1, N))


def _combine_y(a0, a1, sl, wcat, bm):
    M, K = sl.shape
    NY = wcat.shape[-1]

    def body(a0r, a1r, slr, wr, f_ref, y_ref):
        f = a0r[...] + a1r[...] + slr[...]
        f_ref[...] = f
        y_ref[...] = jnp.dot(f, wr[...], preferred_element_type=jnp.float32)

    return pl.pallas_call(
        body,
        grid=(M // bm,),
        in_specs=[
            pl.BlockSpec((bm, K), lambda i: (i, 0)),
            pl.BlockSpec((bm, K), lambda i: (i, 0)),
            pl.BlockSpec((bm, K), lambda i: (i, 0)),
            pl.BlockSpec((K, NY), lambda i: (0, 0)),
        ],
        out_specs=[
            pl.BlockSpec((bm, K), lambda i: (i, 0)),
            pl.BlockSpec((bm, NY), lambda i: (i, 0)),
        ],
        out_shape=[
            jax.ShapeDtypeStruct((M, K), jnp.float32),
            jax.ShapeDtypeStruct((M, NY), jnp.float32),
        ],
    )(a0, a1, sl, wcat)


def _decode(a0, a1, sl, dW1, db1, dW2, db2, ceW1, ceb1, ceW2, ceb2,
            cdW1, cdb1, cdW2, cdb2, bm):
    half = a0.shape[0] // 2
    ng = half // bm
    nclass = dW2.shape[-1]

    def body(a0a, a1a, sla, a0b, a1b, slb, dw1, db1r, dw2, db2r,
             cw1, cb1, cw2, cb2, kw1, kb1, kw2, kb2, res_ref, loss_ref):
        def dec(x):
            h = _leaky(jnp.dot(x, dw1[...], preferred_element_type=jnp.float32) + db1r[...])
            return jnp.dot(h, dw2[...], preferred_element_type=jnp.float32) + db2r[...]

        def constr(f):
            h = _leaky(jnp.dot(f, cw1[...], preferred_element_type=jnp.float32) + cb1[...])
            c = jnp.dot(h, cw2[...], preferred_element_type=jnp.float32) + cb2[...]
            h2 = _leaky(jnp.dot(c, kw1[...], preferred_element_type=jnp.float32) + kb1[...])
            return jnp.dot(h2, kw2[...], preferred_element_type=jnp.float32) + kb2[...]

        fa = dec(a0a[...] + a1a[...] + sla[...])
        fb = dec(a0b[...] + a1b[...] + slb[...])
        ca = constr(fa)
        cb = constr(fb)
        res_ref[...] = 0.5 * (ca + cb)
        part = jnp.sum((ca - fa) ** 2) + jnp.sum((cb - fb) ** 2)
        i = pl.program_id(0)

        @pl.when(i == 0)
        def _():
            loss_ref[...] = jnp.zeros((1, 1), jnp.float32)

        loss_ref[...] += part.reshape(1, 1)

        @pl.when(i == ng - 1)
        def _():
            loss_ref[...] = loss_ref[...] / (2.0 * half * nclass)

    def full(shape):
        return [pl.BlockSpec(shape, lambda i: (0, 0))]

    res, loss = pl.pallas_call(
        body,
        grid=(ng,),
        in_specs=(
            [pl.BlockSpec((bm, a0.shape[1]), lambda i: (i, 0)),
             pl.BlockSpec((bm, a0.shape[1]), lambda i: (i, 0)),
             pl.BlockSpec((bm, a0.shape[1]), lambda i: (i, 0)),
             pl.BlockSpec((bm, a0.shape[1]), lambda i: (i + ng, 0)),
             pl.BlockSpec((bm, a0.shape[1]), lambda i: (i + ng, 0)),
             pl.BlockSpec((bm, a0.shape[1]), lambda i: (i + ng, 0))]
            + full(dW1.shape) + full((1, dW1.shape[1]))
            + full(dW2.shape) + full((1, dW2.shape[1]))
            + full(ceW1.shape) + full((1, ceW1.shape[1]))
            + full(ceW2.shape) + full((1, ceW2.shape[1]))
            + full(cdW1.shape) + full((1, cdW1.shape[1]))
            + full(cdW2.shape) + full((1, cdW2.shape[1]))
        ),
        out_specs=[
            pl.BlockSpec((bm, nclass), lambda i: (i, 0)),
            pl.BlockSpec((1, 1), lambda i: (0, 0)),
        ],
        out_shape=[
            jax.ShapeDtypeStruct((half, nclass), jnp.float32),
            jax.ShapeDtypeStruct((1, 1), jnp.float32),
        ],
    )(a0, a1, sl, a0, a1, sl, dW1, db1.reshape(1, -1), dW2, db2.reshape(1, -1),
      ceW1, ceb1.reshape(1, -1), ceW2, ceb2.reshape(1, -1),
      cdW1, cdb1.reshape(1, -1), cdW2, cdb2.reshape(1, -1))
    return res, loss[0, 0]


def kernel(feat0, feat1, edge_index, etypes,
           e0_W1, e0_b1, e0_W2, e0_b2,
           e1_W1, e1_b1, e1_W2, e1_b2,
           rel_W, self_W, rel_b,
           dec_W1, dec_b1, dec_W2, dec_b2,
           ce_W1, ce_b1, ce_W2, ce_b2,
           cd_W1, cd_b1, cd_W2, cd_b2):
    h0 = _mlp2(feat0, e0_W1, e0_b1, e0_W2, e0_b2, bm=1000)
    h1 = _mlp2(feat1, e1_W1, e1_b1, e1_W2, e1_b2, bm=1000)
    feat = jnp.concatenate([h0, h1], axis=0)

    src = edge_index[0].astype(jnp.int32)
    dst = edge_index[1].astype(jnp.int32)
    et = etypes.astype(jnp.int32)
    pk = _edge_prep(src.reshape(_E // _H, _H), et.reshape(_E // _H, _H),
                    dst.reshape(_E // _H, _H))
    pad = _EPWP - _EPW
    padblk = jnp.broadcast_to(_N + (jnp.arange(pad, dtype=jnp.int32) % 16),
                              (_NW, pad))
    pk = jnp.concatenate([pk.reshape(_NW, _EPW), padblk], axis=1).reshape(-1)
    zin = jnp.zeros((_C, _H), jnp.float32)

    wcat0 = jnp.transpose(rel_W[0], (1, 0, 2)).reshape(_H, _R * _H)
    wcat1 = jnp.transpose(rel_W[1], (1, 0, 2)).reshape(_H, _R * _H)

    y0 = _lin(feat, wcat0, bm=2000).reshape(_N * _R, _H)
    agg_0 = _sc_agg(y0, pk, zin)
    sl0 = _linb(feat, self_W[0], rel_b[0], bm=2000)

    feat1, y1 = _combine_y(agg_0[0], agg_0[1], sl0, wcat1, bm=2000)
    agg_1 = _sc_agg(y1.reshape(_N * _R, _H), pk, zin)
    sl1 = _linb(feat1, self_W[1], rel_b[1], bm=2000)

    res, loss = _decode(agg_1[0], agg_1[1], sl1,
                        dec_W1, dec_b1, dec_W2, dec_b2,
                        ce_W1, ce_b1, ce_W2, ce_b2,
                        cd_W1, cd_b1, cd_W2, cd_b2, bm=1000)
    return res, loss


# speedup vs baseline: 1.4362x; 1.4355x over previous
import functools
import jax
import jax.numpy as jnp
from jax import lax
from jax.experimental import pallas as pl
from jax.experimental.pallas import tpu as pltpu
from jax.experimental.pallas import tpu_sc as plsc

_N = 10000
_H = 128
_E = 320000
_R = 4
_NC = 2
_NS = 16
_NW = _NC * _NS
_EPW = _E // _NW
_C = 80
_EPWP = 10080
_NCHP = _EPWP // _C
_AGGR = 10016
_ZR = 624
_ZC = 80
_DSHIFT = 14


def _sc_agg_body(ycat, pk, zin, out,
                 pk_v, gi0, gi1, db0, db1, rb0, rb1, agg,
                 semE, sem0, sem1):
    cid = lax.axis_index("c")
    sid = lax.axis_index("s")
    wid = sid * _NC + cid
    ebase = wid * _EPWP

    stg = pltpu.async_copy(pk.at[pl.ds(ebase, _EPWP)], pk_v, semE)

    pltpu.sync_copy(zin, rb0)
    zbase = sid * _ZR
    for t in range(7):
        pltpu.sync_copy(rb0.at[pl.ds(0, _ZC)], agg.at[pl.ds(zbase + t * _ZC, _ZC)])
    pltpu.sync_copy(rb0.at[pl.ds(0, 64)], agg.at[pl.ds(zbase + 560, 64)])

    @pl.when(sid == _NS - 1)
    def _():
        pltpu.sync_copy(rb0.at[pl.ds(0, 32)], agg.at[pl.ds(_NS * _ZR, 32)])

    plsc.subcore_barrier()
    stg.wait()

    def _prep(j, gi, db):
        base = j * _C
        for k in range(_C // 16):
            s = pl.ds(base + k * 16, 16)
            w = pk_v[s]
            gi[pl.ds(k * 16, 16)] = lax.shift_right_logical(w, _DSHIFT)
            db[pl.ds(k * 16, 16)] = lax.bitwise_and(w, (1 << _DSHIFT) - 1)

    def _fire(gi, rb, sem):
        pltpu.async_copy(ycat.at[gi], rb, sem)

    def _wait(gi, rb, sem):
        pltpu.make_async_copy(ycat.at[gi], rb, sem).wait()

    _prep(0, gi0, db0)
    _fire(gi0, rb0, sem0)

    def _pair(jj, carry):
        j = jj * 2
        _prep(j + 1, gi1, db1)
        _fire(gi1, rb1, sem1)
        _wait(gi0, rb0, sem0)
        pltpu.sync_copy(rb0, agg.at[db0], add=True)

        @pl.when(j + 2 < _NCHP)
        def _():
            _prep(j + 2, gi0, db0)
            _fire(gi0, rb0, sem0)

        _wait(gi1, rb1, sem1)
        pltpu.sync_copy(rb1, agg.at[db1], add=True)
        return carry

    lax.fori_loop(0, _NCHP // 2, _pair, 0)

    plsc.subcore_barrier()

    RB = (rb0, rb1)
    SD = (sem0, sem1)
    descs = []
    for t in range(8):
        b = t % 2
        rows = _ZC if t < 7 else 64
        if t >= 2:
            descs[t - 2].wait()
        pltpu.sync_copy(agg.at[pl.ds(zbase + t * _ZC, rows)],
                        RB[b].at[pl.ds(0, rows)])
        descs.append(pltpu.async_copy(
            RB[b].at[pl.ds(0, rows)],
            out.at[cid, pl.ds(zbase + t * _ZC, rows)], SD[b]))
    descs[6].wait()
    descs[7].wait()

    @pl.when(sid == _NS - 1)
    def _():
        pltpu.sync_copy(agg.at[pl.ds(_NS * _ZR, 16)], rb0.at[pl.ds(0, 16)])
        pltpu.sync_copy(rb0.at[pl.ds(0, 16)], out.at[cid, pl.ds(_NS * _ZR, 16)])


_sc_agg = pl.kernel(
    _sc_agg_body,
    out_type=jax.ShapeDtypeStruct((_NC, _N, _H), jnp.float32),
    mesh=plsc.VectorSubcoreMesh(
        core_axis_name="c", subcore_axis_name="s",
        num_cores=_NC, num_subcores=_NS),
    scratch_types=[
        pltpu.VMEM((_EPWP,), jnp.int32),
        pltpu.VMEM((_C,), jnp.int32),
        pltpu.VMEM((_C,), jnp.int32),
        pltpu.VMEM((_C,), jnp.int32),
        pltpu.VMEM((_C,), jnp.int32),
        pltpu.VMEM((_C, _H), jnp.float32),
        pltpu.VMEM((_C, _H), jnp.float32),
        pltpu.VMEM_SHARED((_AGGR, _H), jnp.float32),
        pltpu.SemaphoreType.DMA,
        pltpu.SemaphoreType.DMA,
        pltpu.SemaphoreType.DMA,
    ],
)


def _leaky(x):
    return jnp.where(x > 0, x, 0.3 * x)


def _mlp2(x, W1, b1, W2, b2, bm):
    M, K = x.shape
    N1 = W1.shape[-1]
    N2 = W2.shape[-1]

    def body(xr, w1r, b1r, w2r, b2r, orf):
        h = jnp.dot(xr[...], w1r[...], preferred_element_type=jnp.float32)
        h = _leaky(h + b1r[...])
        orf[...] = jnp.dot(h, w2r[...], preferred_element_type=jnp.float32) + b2r[...]

    return pl.pallas_call(
        body,
        grid=(M // bm,),
        in_specs=[
            pl.BlockSpec((bm, K), lambda i: (i, 0)),
            pl.BlockSpec((K, N1), lambda i: (0, 0)),
            pl.BlockSpec((1, N1), lambda i: (0, 0)),
            pl.BlockSpec((N1, N2), lambda i: (0, 0)),
            pl.BlockSpec((1, N2), lambda i: (0, 0)),
        ],
        out_specs=pl.BlockSpec((bm, N2), lambda i: (i, 0)),
        out_shape=jax.ShapeDtypeStruct((M, N2), jnp.float32),
    )(x, W1, b1.reshape(1, N1), W2, b2.reshape(1, N2))


def _lin(x, W, bm):
    M, K = x.shape
    N = W.shape[-1]

    def body(xr, wr, orf):
        orf[...] = jnp.dot(xr[...], wr[...], preferred_element_type=jnp.float32)

    return pl.pallas_call(
        body,
        grid=(M // bm,),
        in_specs=[
            pl.BlockSpec((bm, K), lambda i: (i, 0)),
            pl.BlockSpec((K, N), lambda i: (0, 0)),
        ],
        out_specs=pl.BlockSpec((bm, N), lambda i: (i, 0)),
        out_shape=jax.ShapeDtypeStruct((M, N), jnp.float32),
    )(x, W)


def _edge_prep(src2d, et2d, dst2d):
    rows, cols = src2d.shape

    def body(sr, er, dr, orf):
        gi = sr[...] * _R + er[...]
        orf[...] = jnp.bitwise_or(jnp.left_shift(gi, _DSHIFT), dr[...])

    return pl.pallas_call(
        body,
        grid=(1,),
        in_specs=[
            pl.BlockSpec((rows, cols), lambda i: (0, 0)),
            pl.BlockSpec((rows, cols), lambda i: (0, 0)),
            pl.BlockSpec((rows, cols), lambda i: (0, 0)),
        ],
        out_specs=pl.BlockSpec((rows, cols), lambda i: (0, 0)),
        out_shape=jax.ShapeDtypeStruct((rows, cols), jnp.int32),
    )(src2d, et2d, dst2d)


def _linb(x, W, b, bm):
    M, K = x.shape
    N = W.shape[-1]

    def body(xr, wr, br, orf):
        orf[...] = jnp.dot(xr[...], wr[...], preferred_element_type=jnp.float32) + br[...]

    return pl.pallas_call(
        body,
        grid=(M // bm,),
        in_specs=[
            pl.BlockSpec((bm, K), lambda i: (i, 0)),
            pl.BlockSpec((K, N), lambda i: (0, 0)),
            pl.BlockSpec((1, N), lambda i: (0, 0)),
        ],
        out_specs=pl.BlockSpec((bm, N), lambda i: (i, 0)),
        out_shape=jax.ShapeDtypeStruct((M, N), jnp.float32),
    )(x, W, b.reshape(1, N))


def _combine_y(a0, a1, sl, wcat, bm):
    M, K = sl.shape
    NY = wcat.shape[-1]

    def body(a0r, a1r, slr, wr, f_ref, y_ref):
        f = a0r[...] + a1r[...] + slr[...]
        f_ref[...] = f
        y_ref[...] = jnp.dot(f, wr[...], preferred_element_type=jnp.float32)

    return pl.pallas_call(
        body,
        grid=(M // bm,),
        in_specs=[
            pl.BlockSpec((bm, K), lambda i: (i, 0)),
            pl.BlockSpec((bm, K), lambda i: (i, 0)),
            pl.BlockSpec((bm, K), lambda i: (i, 0)),
            pl.BlockSpec((K, NY), lambda i: (0, 0)),
        ],
        out_specs=[
            pl.BlockSpec((bm, K), lambda i: (i, 0)),
            pl.BlockSpec((bm, NY), lambda i: (i, 0)),
        ],
        out_shape=[
            jax.ShapeDtypeStruct((M, K), jnp.float32),
            jax.ShapeDtypeStruct((M, NY), jnp.float32),
        ],
    )(a0, a1, sl, wcat)


def _decode(a0, a1, sl, dW1, db1, dW2, db2, ceW1, ceb1, ceW2, ceb2,
            cdW1, cdb1, cdW2, cdb2, bm):
    half = a0.shape[0] // 2
    ng = half // bm
    nclass = dW2.shape[-1]

    def body(a0a, a1a, sla, a0b, a1b, slb, dw1, db1r, dw2, db2r,
             cw1, cb1, cw2, cb2, kw1, kb1, kw2, kb2, res_ref, loss_ref):
        def dec(x):
            h = _leaky(jnp.dot(x, dw1[...], preferred_element_type=jnp.float32) + db1r[...])
            return jnp.dot(h, dw2[...], preferred_element_type=jnp.float32) + db2r[...]

        def constr(f):
            h = _leaky(jnp.dot(f, cw1[...], preferred_element_type=jnp.float32) + cb1[...])
            c = jnp.dot(h, cw2[...], preferred_element_type=jnp.float32) + cb2[...]
            h2 = _leaky(jnp.dot(c, kw1[...], preferred_element_type=jnp.float32) + kb1[...])
            return jnp.dot(h2, kw2[...], preferred_element_type=jnp.float32) + kb2[...]

        fa = dec(a0a[...] + a1a[...] + sla[...])
        fb = dec(a0b[...] + a1b[...] + slb[...])
        ca = constr(fa)
        cb = constr(fb)
        res_ref[...] = 0.5 * (ca + cb)
        part = jnp.sum((ca - fa) ** 2) + jnp.sum((cb - fb) ** 2)
        i = pl.program_id(0)

        @pl.when(i == 0)
        def _():
            loss_ref[...] = jnp.zeros((1, 1), jnp.float32)

        loss_ref[...] += part.reshape(1, 1)

        @pl.when(i == ng - 1)
        def _():
            loss_ref[...] = loss_ref[...] / (2.0 * half * nclass)

    def full(shape):
        return [pl.BlockSpec(shape, lambda i: (0, 0))]

    res, loss = pl.pallas_call(
        body,
        grid=(ng,),
        in_specs=(
            [pl.BlockSpec((bm, a0.shape[1]), lambda i: (i, 0)),
             pl.BlockSpec((bm, a0.shape[1]), lambda i: (i, 0)),
             pl.BlockSpec((bm, a0.shape[1]), lambda i: (i, 0)),
             pl.BlockSpec((bm, a0.shape[1]), lambda i: (i + ng, 0)),
             pl.BlockSpec((bm, a0.shape[1]), lambda i: (i + ng, 0)),
             pl.BlockSpec((bm, a0.shape[1]), lambda i: (i + ng, 0))]
            + full(dW1.shape) + full((1, dW1.shape[1]))
            + full(dW2.shape) + full((1, dW2.shape[1]))
            + full(ceW1.shape) + full((1, ceW1.shape[1]))
            + full(ceW2.shape) + full((1, ceW2.shape[1]))
            + full(cdW1.shape) + full((1, cdW1.shape[1]))
            + full(cdW2.shape) + full((1, cdW2.shape[1]))
        ),
        out_specs=[
            pl.BlockSpec((bm, nclass), lambda i: (i, 0)),
            pl.BlockSpec((1, 1), lambda i: (0, 0)),
        ],
        out_shape=[
            jax.ShapeDtypeStruct((half, nclass), jnp.float32),
            jax.ShapeDtypeStruct((1, 1), jnp.float32),
        ],
    )(a0, a1, sl, a0, a1, sl, dW1, db1.reshape(1, -1), dW2, db2.reshape(1, -1),
      ceW1, ceb1.reshape(1, -1), ceW2, ceb2.reshape(1, -1),
      cdW1, cdb1.reshape(1, -1), cdW2, cdb2.reshape(1, -1))
    return res, loss[0, 0]


def kernel(feat0, feat1, edge_index, etypes,
           e0_W1, e0_b1, e0_W2, e0_b2,
           e1_W1, e1_b1, e1_W2, e1_b2,
           rel_W, self_W, rel_b,
           dec_W1, dec_b1, dec_W2, dec_b2,
           ce_W1, ce_b1, ce_W2, ce_b2,
           cd_W1, cd_b1, cd_W2, cd_b2):
    h0 = _mlp2(feat0, e0_W1, e0_b1, e0_W2, e0_b2, bm=1000)
    h1 = _mlp2(feat1, e1_W1, e1_b1, e1_W2, e1_b2, bm=1000)
    feat = jnp.concatenate([h0, h1], axis=0)

    src = edge_index[0].astype(jnp.int32)
    dst = edge_index[1].astype(jnp.int32)
    et = etypes.astype(jnp.int32)
    pk = _edge_prep(src.reshape(_E // _H, _H), et.reshape(_E // _H, _H),
                    dst.reshape(_E // _H, _H))
    pad = _EPWP - _EPW
    padk = jnp.arange(pad, dtype=jnp.int32) % 16
    padblk = jnp.broadcast_to(
        jnp.left_shift(padk * 2048, _DSHIFT) + _N + padk, (_NW, pad))
    pk = jnp.concatenate([pk.reshape(_NW, _EPW), padblk], axis=1).reshape(-1)
    zin = jnp.zeros((_C, _H), jnp.float32)

    wcat0 = jnp.transpose(rel_W[0], (1, 0, 2)).reshape(_H, _R * _H)
    wcat1 = jnp.transpose(rel_W[1], (1, 0, 2)).reshape(_H, _R * _H)

    y0 = _lin(feat, wcat0, bm=2000).reshape(_N * _R, _H)
    agg_0 = _sc_agg(y0, pk, zin)
    sl0 = _linb(feat, self_W[0], rel_b[0], bm=2000)

    feat1, y1 = _combine_y(agg_0[0], agg_0[1], sl0, wcat1, bm=2000)
    agg_1 = _sc_agg(y1.reshape(_N * _R, _H), pk, zin)
    sl1 = _linb(feat1, self_W[1], rel_b[1], bm=2000)

    res, loss = _decode(agg_1[0], agg_1[1], sl1,
                        dec_W1, dec_b1, dec_W2, dec_b2,
                        ce_W1, ce_b1, ce_W2, ce_b2,
                        cd_W1, cd_b1, cd_W2, cd_b2, bm=1000)
    return res, loss


# speedup vs baseline: 1.6086x; 1.1201x over previous
import functools
import jax
import jax.numpy as jnp
from jax import lax
from jax.experimental import pallas as pl
from jax.experimental.pallas import tpu as pltpu
from jax.experimental.pallas import tpu_sc as plsc

_N = 10000
_H = 128
_E = 320000
_R = 4
_NC = 2
_NS = 16
_NW = _NC * _NS
_EPW = _E // _NW
_C = 80
_EPWP = 10080
_NCHP = _EPWP // _C
_AGGR = 10016
_ZR = 624
_ZC = 80
_DSHIFT = 14


def _sc_agg_body(ycat, pk, zin, out,
                 pk_v, gi0, gi1, gi2, db0, db1, db2, rb0, rb1, rb2, agg,
                 semE, sem0, sem1, sem2):
    cid = lax.axis_index("c")
    sid = lax.axis_index("s")
    wid = sid * _NC + cid
    ebase = wid * _EPWP

    stg = pltpu.async_copy(pk.at[pl.ds(ebase, _EPWP)], pk_v, semE)

    pltpu.sync_copy(zin, rb0)
    zbase = sid * _ZR
    for t in range(7):
        pltpu.sync_copy(rb0.at[pl.ds(0, _ZC)], agg.at[pl.ds(zbase + t * _ZC, _ZC)])
    pltpu.sync_copy(rb0.at[pl.ds(0, 64)], agg.at[pl.ds(zbase + 560, 64)])

    @pl.when(sid == _NS - 1)
    def _():
        pltpu.sync_copy(rb0.at[pl.ds(0, 32)], agg.at[pl.ds(_NS * _ZR, 32)])

    plsc.subcore_barrier()
    stg.wait()

    def _prep(j, gi, db):
        base = j * _C
        for k in range(_C // 16):
            s = pl.ds(base + k * 16, 16)
            w = pk_v[s]
            gi[pl.ds(k * 16, 16)] = lax.shift_right_logical(w, _DSHIFT)
            db[pl.ds(k * 16, 16)] = lax.bitwise_and(w, (1 << _DSHIFT) - 1)

    def _fire(gi, rb, sem):
        pltpu.async_copy(ycat.at[gi], rb, sem)

    def _wait(gi, rb, sem):
        pltpu.make_async_copy(ycat.at[gi], rb, sem).wait()

    _prep(0, gi0, db0)
    _fire(gi0, rb0, sem0)
    _prep(1, gi1, db1)
    _fire(gi1, rb1, sem1)

    def _trip(jj, carry):
        j = jj * 3
        _prep(j + 2, gi2, db2)
        _fire(gi2, rb2, sem2)
        _wait(gi0, rb0, sem0)
        pltpu.sync_copy(rb0, agg.at[db0], add=True)

        @pl.when(j + 3 < _NCHP)
        def _():
            _prep(j + 3, gi0, db0)
            _fire(gi0, rb0, sem0)

        _wait(gi1, rb1, sem1)
        pltpu.sync_copy(rb1, agg.at[db1], add=True)

        @pl.when(j + 4 < _NCHP)
        def _():
            _prep(j + 4, gi1, db1)
            _fire(gi1, rb1, sem1)

        _wait(gi2, rb2, sem2)
        pltpu.sync_copy(rb2, agg.at[db2], add=True)
        return carry

    lax.fori_loop(0, _NCHP // 3, _trip, 0)

    plsc.subcore_barrier()

    RB = (rb0, rb1)
    SD = (sem0, sem1)
    descs = []
    for t in range(8):
        b = t % 2
        rows = _ZC if t < 7 else 64
        if t >= 2:
            descs[t - 2].wait()
        pltpu.sync_copy(agg.at[pl.ds(zbase + t * _ZC, rows)],
                        RB[b].at[pl.ds(0, rows)])
        descs.append(pltpu.async_copy(
            RB[b].at[pl.ds(0, rows)],
            out.at[cid, pl.ds(zbase + t * _ZC, rows)], SD[b]))
    descs[6].wait()
    descs[7].wait()

    @pl.when(sid == _NS - 1)
    def _():
        pltpu.sync_copy(agg.at[pl.ds(_NS * _ZR, 16)], rb0.at[pl.ds(0, 16)])
        pltpu.sync_copy(rb0.at[pl.ds(0, 16)], out.at[cid, pl.ds(_NS * _ZR, 16)])


_sc_agg = pl.kernel(
    _sc_agg_body,
    out_type=jax.ShapeDtypeStruct((_NC, _N, _H), jnp.float32),
    mesh=plsc.VectorSubcoreMesh(
        core_axis_name="c", subcore_axis_name="s",
        num_cores=_NC, num_subcores=_NS),
    scratch_types=[
        pltpu.VMEM((_EPWP,), jnp.int32),
        pltpu.VMEM((_C,), jnp.int32),
        pltpu.VMEM((_C,), jnp.int32),
        pltpu.VMEM((_C,), jnp.int32),
        pltpu.VMEM((_C,), jnp.int32),
        pltpu.VMEM((_C,), jnp.int32),
        pltpu.VMEM((_C,), jnp.int32),
        pltpu.VMEM((_C, _H), jnp.float32),
        pltpu.VMEM((_C, _H), jnp.float32),
        pltpu.VMEM((_C, _H), jnp.float32),
        pltpu.VMEM_SHARED((_AGGR, _H), jnp.float32),
        pltpu.SemaphoreType.DMA,
        pltpu.SemaphoreType.DMA,
        pltpu.SemaphoreType.DMA,
        pltpu.SemaphoreType.DMA,
    ],
)


def _leaky(x):
    return jnp.where(x > 0, x, 0.3 * x)


def _mlp2(x, W1, b1, W2, b2, bm):
    M, K = x.shape
    N1 = W1.shape[-1]
    N2 = W2.shape[-1]

    def body(xr, w1r, b1r, w2r, b2r, orf):
        h = jnp.dot(xr[...], w1r[...], preferred_element_type=jnp.float32)
        h = _leaky(h + b1r[...])
        orf[...] = jnp.dot(h, w2r[...], preferred_element_type=jnp.float32) + b2r[...]

    return pl.pallas_call(
        body,
        grid=(M // bm,),
        in_specs=[
            pl.BlockSpec((bm, K), lambda i: (i, 0)),
            pl.BlockSpec((K, N1), lambda i: (0, 0)),
            pl.BlockSpec((1, N1), lambda i: (0, 0)),
            pl.BlockSpec((N1, N2), lambda i: (0, 0)),
            pl.BlockSpec((1, N2), lambda i: (0, 0)),
        ],
        out_specs=pl.BlockSpec((bm, N2), lambda i: (i, 0)),
        out_shape=jax.ShapeDtypeStruct((M, N2), jnp.float32),
    )(x, W1, b1.reshape(1, N1), W2, b2.reshape(1, N2))


def _lin(x, W, bm):
    M, K = x.shape
    N = W.shape[-1]

    def body(xr, wr, orf):
        orf[...] = jnp.dot(xr[...], wr[...], preferred_element_type=jnp.float32)

    return pl.pallas_call(
        body,
        grid=(M // bm,),
        in_specs=[
            pl.BlockSpec((bm, K), lambda i: (i, 0)),
            pl.BlockSpec((K, N), lambda i: (0, 0)),
        ],
        out_specs=pl.BlockSpec((bm, N), lambda i: (i, 0)),
        out_shape=jax.ShapeDtypeStruct((M, N), jnp.float32),
    )(x, W)


def _edge_prep(src2d, et2d, dst2d):
    rows, cols = src2d.shape

    def body(sr, er, dr, orf):
        gi = sr[...] * _R + er[...]
        orf[...] = jnp.bitwise_or(jnp.left_shift(gi, _DSHIFT), dr[...])

    return pl.pallas_call(
        body,
        grid=(1,),
        in_specs=[
            pl.BlockSpec((rows, cols), lambda i: (0, 0)),
            pl.BlockSpec((rows, cols), lambda i: (0, 0)),
            pl.BlockSpec((rows, cols), lambda i: (0, 0)),
        ],
        out_specs=pl.BlockSpec((rows, cols), lambda i: (0, 0)),
        out_shape=jax.ShapeDtypeStruct((rows, cols), jnp.int32),
    )(src2d, et2d, dst2d)


def _linb(x, W, b, bm):
    M, K = x.shape
    N = W.shape[-1]

    def body(xr, wr, br, orf):
        orf[...] = jnp.dot(xr[...], wr[...], preferred_element_type=jnp.float32) + br[...]

    return pl.pallas_call(
        body,
        grid=(M // bm,),
        in_specs=[
            pl.BlockSpec((bm, K), lambda i: (i, 0)),
            pl.BlockSpec((K, N), lambda i: (0, 0)),
            pl.BlockSpec((1, N), lambda i: (0, 0)),
        ],
        out_specs=pl.BlockSpec((bm, N), lambda i: (i, 0)),
        out_shape=jax.ShapeDtypeStruct((M, N), jnp.float32),
    )(x, W, b.reshape(1, N))


def _combine_y(a0, a1, sl, wcat, bm):
    M, K = sl.shape
    NY = wcat.shape[-1]

    def body(a0r, a1r, slr, wr, f_ref, y_ref):
        f = a0r[...] + a1r[...] + slr[...]
        f_ref[...] = f
        y_ref[...] = jnp.dot(f, wr[...], preferred_element_type=jnp.float32)

    return pl.pallas_call(
        body,
        grid=(M // bm,),
        in_specs=[
            pl.BlockSpec((bm, K), lambda i: (i, 0)),
            pl.BlockSpec((bm, K), lambda i: (i, 0)),
            pl.BlockSpec((bm, K), lambda i: (i, 0)),
            pl.BlockSpec((K, NY), lambda i: (0, 0)),
        ],
        out_specs=[
            pl.BlockSpec((bm, K), lambda i: (i, 0)),
            pl.BlockSpec((bm, NY), lambda i: (i, 0)),
        ],
        out_shape=[
            jax.ShapeDtypeStruct((M, K), jnp.float32),
            jax.ShapeDtypeStruct((M, NY), jnp.float32),
        ],
    )(a0, a1, sl, wcat)


def _decode(a0, a1, sl, dW1, db1, dW2, db2, ceW1, ceb1, ceW2, ceb2,
            cdW1, cdb1, cdW2, cdb2, bm):
    half = a0.shape[0] // 2
    ng = half // bm
    nclass = dW2.shape[-1]

    def body(a0a, a1a, sla, a0b, a1b, slb, dw1, db1r, dw2, db2r,
             cw1, cb1, cw2, cb2, kw1, kb1, kw2, kb2, res_ref, loss_ref):
        def dec(x):
            h = _leaky(jnp.dot(x, dw1[...], preferred_element_type=jnp.float32) + db1r[...])
            return jnp.dot(h, dw2[...], preferred_element_type=jnp.float32) + db2r[...]

        def constr(f):
            h = _leaky(jnp.dot(f, cw1[...], preferred_element_type=jnp.float32) + cb1[...])
            c = jnp.dot(h, cw2[...], preferred_element_type=jnp.float32) + cb2[...]
            h2 = _leaky(jnp.dot(c, kw1[...], preferred_element_type=jnp.float32) + kb1[...])
            return jnp.dot(h2, kw2[...], preferred_element_type=jnp.float32) + kb2[...]

        fa = dec(a0a[...] + a1a[...] + sla[...])
        fb = dec(a0b[...] + a1b[...] + slb[...])
        ca = constr(fa)
        cb = constr(fb)
        res_ref[...] = 0.5 * (ca + cb)
        part = jnp.sum((ca - fa) ** 2) + jnp.sum((cb - fb) ** 2)
        i = pl.program_id(0)

        @pl.when(i == 0)
        def _():
            loss_ref[...] = jnp.zeros((1, 1), jnp.float32)

        loss_ref[...] += part.reshape(1, 1)

        @pl.when(i == ng - 1)
        def _():
            loss_ref[...] = loss_ref[...] / (2.0 * half * nclass)

    def full(shape):
        return [pl.BlockSpec(shape, lambda i: (0, 0))]

    res, loss = pl.pallas_call(
        body,
        grid=(ng,),
        in_specs=(
            [pl.BlockSpec((bm, a0.shape[1]), lambda i: (i, 0)),
             pl.BlockSpec((bm, a0.shape[1]), lambda i: (i, 0)),
             pl.BlockSpec((bm, a0.shape[1]), lambda i: (i, 0)),
             pl.BlockSpec((bm, a0.shape[1]), lambda i: (i + ng, 0)),
             pl.BlockSpec((bm, a0.shape[1]), lambda i: (i + ng, 0)),
             pl.BlockSpec((bm, a0.shape[1]), lambda i: (i + ng, 0))]
            + full(dW1.shape) + full((1, dW1.shape[1]))
            + full(dW2.shape) + full((1, dW2.shape[1]))
            + full(ceW1.shape) + full((1, ceW1.shape[1]))
            + full(ceW2.shape) + full((1, ceW2.shape[1]))
            + full(cdW1.shape) + full((1, cdW1.shape[1]))
            + full(cdW2.shape) + full((1, cdW2.shape[1]))
        ),
        out_specs=[
            pl.BlockSpec((bm, nclass), lambda i: (i, 0)),
            pl.BlockSpec((1, 1), lambda i: (0, 0)),
        ],
        out_shape=[
            jax.ShapeDtypeStruct((half, nclass), jnp.float32),
            jax.ShapeDtypeStruct((1, 1), jnp.float32),
        ],
    )(a0, a1, sl, a0, a1, sl, dW1, db1.reshape(1, -1), dW2, db2.reshape(1, -1),
      ceW1, ceb1.reshape(1, -1), ceW2, ceb2.reshape(1, -1),
      cdW1, cdb1.reshape(1, -1), cdW2, cdb2.reshape(1, -1))
    return res, loss[0, 0]


def kernel(feat0, feat1, edge_index, etypes,
           e0_W1, e0_b1, e0_W2, e0_b2,
           e1_W1, e1_b1, e1_W2, e1_b2,
           rel_W, self_W, rel_b,
           dec_W1, dec_b1, dec_W2, dec_b2,
           ce_W1, ce_b1, ce_W2, ce_b2,
           cd_W1, cd_b1, cd_W2, cd_b2):
    h0 = _mlp2(feat0, e0_W1, e0_b1, e0_W2, e0_b2, bm=1000)
    h1 = _mlp2(feat1, e1_W1, e1_b1, e1_W2, e1_b2, bm=1000)
    feat = jnp.concatenate([h0, h1], axis=0)

    src = edge_index[0].astype(jnp.int32)
    dst = edge_index[1].astype(jnp.int32)
    et = etypes.astype(jnp.int32)
    pk = _edge_prep(src.reshape(_E // _H, _H), et.reshape(_E // _H, _H),
                    dst.reshape(_E // _H, _H))
    pad = _EPWP - _EPW
    padk = jnp.arange(pad, dtype=jnp.int32) % 16
    padblk = jnp.broadcast_to(
        jnp.left_shift(padk * 2048, _DSHIFT) + _N + padk, (_NW, pad))
    pk = jnp.concatenate([pk.reshape(_NW, _EPW), padblk], axis=1).reshape(-1)
    zin = jnp.zeros((_C, _H), jnp.float32)

    wcat0 = jnp.transpose(rel_W[0], (1, 0, 2)).reshape(_H, _R * _H)
    wcat1 = jnp.transpose(rel_W[1], (1, 0, 2)).reshape(_H, _R * _H)

    y0 = _lin(feat, wcat0, bm=2000).reshape(_N * _R, _H)
    agg_0 = _sc_agg(y0, pk, zin)
    sl0 = _linb(feat, self_W[0], rel_b[0], bm=2000)

    feat1, y1 = _combine_y(agg_0[0], agg_0[1], sl0, wcat1, bm=2000)
    agg_1 = _sc_agg(y1.reshape(_N * _R, _H), pk, zin)
    sl1 = _linb(feat1, self_W[1], rel_b[1], bm=2000)

    res, loss = _decode(agg_1[0], agg_1[1], sl1,
                        dec_W1, dec_b1, dec_W2, dec_b2,
                        ce_W1, ce_b1, ce_W2, ce_b2,
                        cd_W1, cd_b1, cd_W2, cd_b2, bm=1000)
    return res, loss
